# Initial kernel scaffold; baseline (speedup 1.0000x reference)
import jax
import jax.numpy as jnp
from jax.experimental import pallas as pl


def kernel(x, pmeans, psigmas, pvalues, bias):
    raise NotImplementedError("write your pallas kernel here")


# speedup vs baseline: 1.9017x; 1.9017x over previous
import jax
import jax.numpy as jnp
from jax.experimental import pallas as pl
from jax.experimental.pallas import tpu as pltpu

_EPS = 1e-6
_B, _C, _H, _W = 2, 96, 224, 224
_O, _K, _KS = 96, 4, 3
_GA, _RA = 2, 2
_T = 8 + _GA + _RA
_SIGMA_BOOST = 2.0
_SIGMA_SCALE = 0.1
_SIZE = (96.0, 3.0, 3.0)
_RR = (20.0, 3.0, 3.0)
_MULT = (1.0, 288.0, 96.0)
_OK = _O * _K
_WPAD = 1024
_WS = 256
_HP = _H + 2
_NF = _H * _WS
_NB = 8192


def _wker_body(pm_ref, ps_ref, pv_ref, u_ref, sel_ref, out_ref):
    lane = jax.lax.broadcasted_iota(jnp.int32, (_OK, _T), 1)
    s = ps_ref[:, 0:1] + _SIGMA_BOOST
    softplus = jnp.maximum(s, 0.0) + jnp.log(1.0 + jnp.exp(-jnp.abs(s)))
    dsum = jnp.zeros((_OK, _T), jnp.float32)
    jidx = jnp.zeros((_OK, _T), jnp.float32)
    for d in range(3):
        size_d, rr_d = _SIZE[d], _RR[d]
        pm = pm_ref[:, d:d + 1]
        m = (1.0 / (1.0 + jnp.exp(-pm))) * (size_d - 1.0)
        sg = softplus * size_d * _SIGMA_SCALE + _EPS
        u = u_ref[:, d * _T:(d + 1) * _T]
        fl = ((7 - lane) >> (2 - d)) & 1
        nb = jnp.where(fl == 1, jnp.floor(m), jnp.ceil(m))
        gv = jnp.floor(u * size_d)
        lower = jnp.clip(jnp.round(m) - rr_d * 0.5, 0.0, size_d - rr_d)
        lv = jnp.floor(u * rr_d + lower)
        v = jnp.where(lane < 8, nb, jnp.where(lane < 10, gv, lv))
        v = jnp.clip(v, 0.0, size_d - 1.0)
        diff = (v - m) * jnp.sqrt(1.0 / (_EPS + sg))
        dsum = dsum + diff * diff
        jidx = jidx + v * _MULT[d]
    dens = jnp.exp(-0.5 * dsum)
    props = dens / (jnp.sum(dens, axis=1, keepdims=True) + _EPS)
    w = props * pv_ref[:, 0:1]
    idx = jidx.astype(jnp.int32)
    lanes2 = jax.lax.broadcasted_iota(jnp.int32, (_OK, _WPAD), 1)
    acc = jnp.zeros((_OK, _WPAD), jnp.float32)
    for t in range(_T):
        acc = acc + jnp.where(lanes2 == idx[:, t:t + 1], w[:, t:t + 1], 0.0)
    out_ref[...] = jnp.dot(sel_ref[...], acc,
                           preferred_element_type=jnp.float32)


def _conv_body(x0_ref, x1_ref, x2_ref, wt_ref, b_ref, out_ref):
    acc = jnp.zeros((_O, _NB), jnp.float32)
    for dy, xr in enumerate((x0_ref, x1_ref, x2_ref)):
        xb = xr[0]
        for dx in range(3):
            xs = xb if dx == 0 else pltpu.roll(xb, _NB - dx, 1)
            acc = acc + jnp.dot(wt_ref[3 * dy + dx], xs,
                                preferred_element_type=jnp.float32)
    out_ref[0] = acc + b_ref[:, 0:1]


def kernel(x, pmeans, psigmas, pvalues, bias):
    f32 = jnp.float32
    kg, kl = jax.random.split(jax.random.key(42))
    gu = jax.random.uniform(kg, (_O, _K, _GA, 3), dtype=f32) * (1.0 - _EPS)
    lu = jax.random.uniform(kl, (_O, _K, _RA, 3), dtype=f32) * (1.0 - _EPS)
    u = jnp.concatenate([jnp.zeros((_O, _K, 8, 3), f32), gu, lu], axis=2)
    upk = jnp.concatenate([u[..., d].reshape(_OK, _T) for d in range(3)],
                          axis=1)
    sel = (jnp.arange(_O)[:, None] == (jnp.arange(_OK)[None, :] // _K))
    sel = sel.astype(f32)

    wflat = pl.pallas_call(
        _wker_body,
        out_shape=jax.ShapeDtypeStruct((_O, _WPAD), f32),
    )(pmeans.reshape(_OK, 3), psigmas.reshape(_OK, 1),
      pvalues.reshape(_OK, 1), upk, sel)
    wt = wflat[:, :_KS * _KS * _C].reshape(_O, _KS * _KS, _C).transpose(1, 0, 2)

    xp = jnp.pad(x, ((0, 0), (0, 0), (1, 1), (1, _WS - _W - 1)))
    xpf = xp.reshape(_B, _C, _HP * _WS)
    xv = [jax.lax.slice_in_dim(xpf, dy * _WS, dy * _WS + _NF, axis=2)
          for dy in range(3)]

    out = pl.pallas_call(
        _conv_body,
        grid=(_B, _NF // _NB),
        in_specs=[pl.BlockSpec((1, _C, _NB), lambda b, i: (b, 0, i))] * 3 + [
            pl.BlockSpec((_KS * _KS, _O, _C), lambda b, i: (0, 0, 0)),
            pl.BlockSpec((_O, 1), lambda b, i: (0, 0)),
        ],
        out_specs=pl.BlockSpec((1, _O, _NB), lambda b, i: (b, 0, i)),
        out_shape=jax.ShapeDtypeStruct((_B, _O, _NF), f32),
    )(xv[0], xv[1], xv[2], wt, bias.reshape(_O, 1))
    return out.reshape(_B, _O, _H, _WS)[:, :, :, :_W]


# speedup vs baseline: 2.1970x; 1.1553x over previous
import jax
import jax.numpy as jnp
from jax.experimental import pallas as pl
from jax.experimental.pallas import tpu as pltpu

_EPS = 1e-6
_B, _C, _H, _W = 2, 96, 224, 224
_O, _K, _KS = 96, 4, 3
_GA, _RA = 2, 2
_T = 8 + _GA + _RA
_SIGMA_BOOST = 2.0
_SIGMA_SCALE = 0.1
_SIZE = (96.0, 3.0, 3.0)
_RR = (20.0, 3.0, 3.0)
_MULT = (1.0, 288.0, 96.0)
_OK = _O * _K
_WPAD = 1024
_WS = 256
_HP = _H + 2
_NF = _H * _WS
_NB = 4096


def _wker_body(pm_ref, ps_ref, pv_ref, u_ref, sel_ref, out_ref):
    lane = jax.lax.broadcasted_iota(jnp.int32, (_OK, _T), 1)
    s = ps_ref[:, 0:1] + _SIGMA_BOOST
    softplus = jnp.maximum(s, 0.0) + jnp.log(1.0 + jnp.exp(-jnp.abs(s)))
    dsum = jnp.zeros((_OK, _T), jnp.float32)
    jidx = jnp.zeros((_OK, _T), jnp.float32)
    for d in range(3):
        size_d, rr_d = _SIZE[d], _RR[d]
        pm = pm_ref[:, d:d + 1]
        m = (1.0 / (1.0 + jnp.exp(-pm))) * (size_d - 1.0)
        sg = softplus * size_d * _SIGMA_SCALE + _EPS
        u = u_ref[:, d * _T:(d + 1) * _T]
        fl = ((7 - lane) >> (2 - d)) & 1
        nb = jnp.where(fl == 1, jnp.floor(m), jnp.ceil(m))
        gv = jnp.floor(u * size_d)
        lower = jnp.clip(jnp.round(m) - rr_d * 0.5, 0.0, size_d - rr_d)
        lv = jnp.floor(u * rr_d + lower)
        v = jnp.where(lane < 8, nb, jnp.where(lane < 10, gv, lv))
        v = jnp.clip(v, 0.0, size_d - 1.0)
        diff = (v - m) * jnp.sqrt(1.0 / (_EPS + sg))
        dsum = dsum + diff * diff
        jidx = jidx + v * _MULT[d]
    dens = jnp.exp(-0.5 * dsum)
    props = dens / (jnp.sum(dens, axis=1, keepdims=True) + _EPS)
    w = props * pv_ref[:, 0:1]
    idx = jidx.astype(jnp.int32)
    lanes2 = jax.lax.broadcasted_iota(jnp.int32, (_OK, _WPAD), 1)
    acc = jnp.zeros((_OK, _WPAD), jnp.float32)
    for t in range(_T):
        acc = acc + jnp.where(lanes2 == idx[:, t:t + 1], w[:, t:t + 1], 0.0)
    out_ref[...] = jnp.dot(sel_ref[...], acc,
                           preferred_element_type=jnp.float32)


def _conv_body(x0_ref, x1_ref, x2_ref, wt_ref, b_ref, out_ref):
    parts = []
    for xr in (x0_ref, x1_ref, x2_ref):
        xb = xr[0]
        for dx in range(3):
            parts.append(xb if dx == 0 else pltpu.roll(xb, _NB - dx, 1))
    xcat = jnp.concatenate(parts, axis=0)
    acc = jnp.dot(wt_ref[...], xcat, preferred_element_type=jnp.float32)
    out_ref[0] = acc + b_ref[:, 0:1]


def kernel(x, pmeans, psigmas, pvalues, bias):
    f32 = jnp.float32
    kg, kl = jax.random.split(jax.random.key(42))
    gu = jax.random.uniform(kg, (_O, _K, _GA, 3), dtype=f32) * (1.0 - _EPS)
    lu = jax.random.uniform(kl, (_O, _K, _RA, 3), dtype=f32) * (1.0 - _EPS)
    u = jnp.concatenate([jnp.zeros((_O, _K, 8, 3), f32), gu, lu], axis=2)
    upk = jnp.concatenate([u[..., d].reshape(_OK, _T) for d in range(3)],
                          axis=1)
    sel = (jnp.arange(_O)[:, None] == (jnp.arange(_OK)[None, :] // _K))
    sel = sel.astype(f32)

    wflat = pl.pallas_call(
        _wker_body,
        out_shape=jax.ShapeDtypeStruct((_O, _WPAD), f32),
    )(pmeans.reshape(_OK, 3), psigmas.reshape(_OK, 1),
      pvalues.reshape(_OK, 1), upk, sel)
    wt = wflat[:, :_KS * _KS * _C].astype(jnp.bfloat16)

    xp = jnp.pad(x, ((0, 0), (0, 0), (1, 1), (1, _WS - _W - 1)))
    xpf = xp.reshape(_B, _C, _HP * _WS).astype(jnp.bfloat16)
    xv = [jax.lax.slice_in_dim(xpf, dy * _WS, dy * _WS + _NF, axis=2)
          for dy in range(3)]

    out = pl.pallas_call(
        _conv_body,
        grid=(_B, _NF // _NB),
        in_specs=[pl.BlockSpec((1, _C, _NB), lambda b, i: (b, 0, i))] * 3 + [
            pl.BlockSpec((_O, _KS * _KS * _C), lambda b, i: (0, 0)),
            pl.BlockSpec((_O, 1), lambda b, i: (0, 0)),
        ],
        out_specs=pl.BlockSpec((1, _O, _NB), lambda b, i: (b, 0, i)),
        out_shape=jax.ShapeDtypeStruct((_B, _O, _NF), f32),
    )(xv[0], xv[1], xv[2], wt, bias.reshape(_O, 1))
    return out.reshape(_B, _O, _H, _WS)[:, :, :, :_W]


# speedup vs baseline: 2.7781x; 1.2645x over previous
import jax
import jax.numpy as jnp
from jax.experimental import pallas as pl
from jax.experimental.pallas import tpu as pltpu

_EPS = 1e-6
_B, _C, _H, _W = 2, 96, 224, 224
_O, _K, _KS = 96, 4, 3
_GA, _RA = 2, 2
_T = 8 + _GA + _RA
_SIGMA_BOOST = 2.0
_SIGMA_SCALE = 0.1
_SIZE = (96.0, 3.0, 3.0)
_RR = (20.0, 3.0, 3.0)
_MULT = (1.0, 288.0, 96.0)
_OK = _O * _K
_WPAD = 1024
_WS = 256
_HP = _H + 2
_NF = _H * _WS
_NB = 4096


def _wker_body(pm_ref, ps_ref, pv_ref, u_ref, sel_ref, out_ref):
    lane = jax.lax.broadcasted_iota(jnp.int32, (_OK, _T), 1)
    s = ps_ref[:, 0:1] + _SIGMA_BOOST
    softplus = jnp.maximum(s, 0.0) + jnp.log(1.0 + jnp.exp(-jnp.abs(s)))
    dsum = jnp.zeros((_OK, _T), jnp.float32)
    jidx = jnp.zeros((_OK, _T), jnp.float32)
    for d in range(3):
        size_d, rr_d = _SIZE[d], _RR[d]
        pm = pm_ref[:, d:d + 1]
        m = (1.0 / (1.0 + jnp.exp(-pm))) * (size_d - 1.0)
        sg = softplus * size_d * _SIGMA_SCALE + _EPS
        u = u_ref[:, d * _T:(d + 1) * _T]
        fl = ((7 - lane) >> (2 - d)) & 1
        nb = jnp.where(fl == 1, jnp.floor(m), jnp.ceil(m))
        gv = jnp.floor(u * size_d)
        lower = jnp.clip(jnp.round(m) - rr_d * 0.5, 0.0, size_d - rr_d)
        lv = jnp.floor(u * rr_d + lower)
        v = jnp.where(lane < 8, nb, jnp.where(lane < 10, gv, lv))
        v = jnp.clip(v, 0.0, size_d - 1.0)
        diff = (v - m) * jnp.sqrt(1.0 / (_EPS + sg))
        dsum = dsum + diff * diff
        jidx = jidx + v * _MULT[d]
    dens = jnp.exp(-0.5 * dsum)
    props = dens / (jnp.sum(dens, axis=1, keepdims=True) + _EPS)
    w = props * pv_ref[:, 0:1]
    idx = jidx.astype(jnp.int32)
    lanes2 = jax.lax.broadcasted_iota(jnp.int32, (_OK, _WPAD), 1)
    acc = jnp.zeros((_OK, _WPAD), jnp.float32)
    for t in range(_T):
        acc = acc + jnp.where(lanes2 == idx[:, t:t + 1], w[:, t:t + 1], 0.0)
    out_ref[...] = jnp.dot(sel_ref[...], acc,
                           preferred_element_type=jnp.float32)


_NI = _NF // _NB
_NS = _B * _NI
_NBH = _NB + 3 * _WS


def _conv_body(xf_hbm, wt_ref, b_ref, out_ref, xbuf, sem):
    b = pl.program_id(0)
    i = pl.program_id(1)
    s = b * _NI + i
    slot = jax.lax.rem(s, 2)

    def _start(step, slot_):
        bb = jax.lax.div(step, _NI)
        ii = jax.lax.rem(step, _NI)
        pltpu.make_async_copy(
            xf_hbm.at[bb, :, pl.ds(ii * _NB, _NBH)],
            xbuf.at[slot_], sem.at[slot_]).start()

    @pl.when(s == 0)
    def _():
        _start(s, slot)

    @pl.when(s + 1 < _NS)
    def _():
        _start(s + 1, 1 - slot)

    pltpu.make_async_copy(
        xf_hbm.at[0, :, pl.ds(0, _NBH)], xbuf.at[slot], sem.at[slot]).wait()

    parts = [xbuf[slot, :, pl.ds(dy * _WS + dx, _NB)]
             for dy in range(3) for dx in range(3)]
    xcat = jnp.concatenate(parts, axis=0)
    acc = jnp.dot(wt_ref[...], xcat, preferred_element_type=jnp.float32)
    out_ref[0] = acc + b_ref[:, 0:1]


def kernel(x, pmeans, psigmas, pvalues, bias):
    f32 = jnp.float32
    kg, kl = jax.random.split(jax.random.key(42))
    gu = jax.random.uniform(kg, (_O, _K, _GA, 3), dtype=f32) * (1.0 - _EPS)
    lu = jax.random.uniform(kl, (_O, _K, _RA, 3), dtype=f32) * (1.0 - _EPS)
    u = jnp.concatenate([jnp.zeros((_O, _K, 8, 3), f32), gu, lu], axis=2)
    upk = jnp.concatenate([u[..., d].reshape(_OK, _T) for d in range(3)],
                          axis=1)
    sel = (jnp.arange(_O)[:, None] == (jnp.arange(_OK)[None, :] // _K))
    sel = sel.astype(f32)

    wflat = pl.pallas_call(
        _wker_body,
        out_shape=jax.ShapeDtypeStruct((_O, _WPAD), f32),
    )(pmeans.reshape(_OK, 3), psigmas.reshape(_OK, 1),
      pvalues.reshape(_OK, 1), upk, sel)
    wt = wflat[:, :_KS * _KS * _C].astype(jnp.bfloat16)

    xp = jnp.pad(x, ((0, 0), (0, 0), (1, 3), (1, _WS - _W - 1)))
    xpf = xp.reshape(_B, _C, (_H + 4) * _WS).astype(jnp.bfloat16)

    out = pl.pallas_call(
        _conv_body,
        grid=(_B, _NI),
        in_specs=[
            pl.BlockSpec(memory_space=pl.MemorySpace.ANY),
            pl.BlockSpec((_O, _KS * _KS * _C), lambda b, i: (0, 0)),
            pl.BlockSpec((_O, 1), lambda b, i: (0, 0)),
        ],
        out_specs=pl.BlockSpec((1, _O, _NB), lambda b, i: (b, 0, i)),
        out_shape=jax.ShapeDtypeStruct((_B, _O, _NF), f32),
        scratch_shapes=[
            pltpu.VMEM((2, _C, _NBH), jnp.bfloat16),
            pltpu.SemaphoreType.DMA((2,)),
        ],
    )(xpf, wt, bias.reshape(_O, 1))
    return out.reshape(_B, _O, _H, _WS)[:, :, :, :_W]
